# baseline (speedup 1.0000x reference)
import jax
import jax.numpy as jnp
from jax.experimental import pallas as pl
from jax.experimental.pallas import tpu as pltpu

_SELECTED = (0, 2, 4, 6, 8, 10, 12, 14)


def _copy_body(in_ref, out_ref, sems):
    for k, s in enumerate(_SELECTED):
        pltpu.make_async_copy(in_ref.at[s], out_ref.at[k], sems.at[k]).start()
    for k, s in enumerate(_SELECTED):
        pltpu.make_async_copy(in_ref.at[s], out_ref.at[k], sems.at[k]).wait()


def kernel(w):
    n_out = len(_SELECTED)
    _, H, W = w.shape
    return pl.pallas_call(
        _copy_body,
        in_specs=[pl.BlockSpec(memory_space=pl.ANY)],
        out_specs=pl.BlockSpec(memory_space=pl.ANY),
        out_shape=jax.ShapeDtypeStruct((n_out, H, W), w.dtype),
        scratch_shapes=[pltpu.SemaphoreType.DMA((n_out,))],
    )(w)


# speedup vs baseline: 49.0074x; 49.0074x over previous
import jax
import jax.numpy as jnp
from jax.experimental import pallas as pl
from jax.experimental.pallas import tpu as pltpu

_SELECTED = (0, 2, 4, 6, 8, 10, 12, 14)
_ROWS = 1024


def _copy_body(in_ref, out_ref):
    out_ref[...] = in_ref[...]


def kernel(w):
    n_out = len(_SELECTED)
    _, H, W = w.shape
    return pl.pallas_call(
        _copy_body,
        grid=(n_out, H // _ROWS),
        in_specs=[pl.BlockSpec((1, _ROWS, W), lambda i, j: (2 * i, j, 0))],
        out_specs=pl.BlockSpec((1, _ROWS, W), lambda i, j: (i, j, 0)),
        out_shape=jax.ShapeDtypeStruct((n_out, H, W), w.dtype),
        compiler_params=pltpu.CompilerParams(
            dimension_semantics=("parallel", "parallel"),
        ),
    )(w)
